# baseline (speedup 1.0000x reference)
import functools

import numpy as np

import jax
import jax.numpy as jnp
from jax import lax
from jax.experimental import pallas as pl
from jax.experimental.pallas import tpu as pltpu
from jax.experimental.pallas import tpu_sc as plsc

_N = 131072
_C = 16384
_NT = 16
_CHUNK = _N // _NT
_NV = _CHUNK // 16
_BINS = 256
_LANEH = 16 * _BINS

_W = np.zeros((2, 15), np.float32)
_W[0, 1:5] = 0.25
_W[1, 5:15] = 0.1


def _tc_losses_body(pred_ref, lab_ref, tgt_ref, keys_ref, cnt_ref):
    i = pl.program_id(0)
    lab = lab_ref[...]
    s = jax.nn.sigmoid(pred_ref[0:1, :])
    y = jnp.where(jnp.logical_xor(lab == 1.0, lab == -2.0),
                  jnp.float32(1.0), jnp.float32(0.0))
    l_cls = jnp.maximum(s, 0.0) - s * y + jnp.log1p(jnp.exp(-jnp.abs(s)))
    m_cls = lab >= 0.0
    d = pred_ref[1:5, :] - tgt_ref[1:5, :]
    per_off = jnp.sum(d * d, axis=0, keepdims=True) / 4.0
    e = pred_ref[5:15, :] - tgt_ref[5:15, :]
    per_lm = jnp.sum(e * e, axis=0, keepdims=True) / 10.0
    m_off = jnp.logical_xor(lab == 1.0, lab == -1.0)
    m_lm = lab == -2.0
    zero = jnp.int32(0)
    k0 = jnp.where(m_cls, lax.bitcast_convert_type(l_cls, jnp.int32), zero)
    k1 = jnp.where(m_off, lax.bitcast_convert_type(per_off, jnp.int32), zero)
    k2 = jnp.where(m_lm, lax.bitcast_convert_type(per_lm, jnp.int32), zero)
    keys_ref[...] = jnp.concatenate([k0, k1, k2], axis=0)

    @pl.when(i == 0)
    def _():
        for j in range(16):
            cnt_ref[j] = jnp.int32(0)

    cnt_ref[0] += jnp.sum(m_cls.astype(jnp.float32)).astype(jnp.int32)
    cnt_ref[1] += jnp.sum(m_off.astype(jnp.float32)).astype(jnp.int32)
    cnt_ref[2] += jnp.sum(m_lm.astype(jnp.float32)).astype(jnp.int32)


def _tc_losses(predT, labR, tgtT):
    grid = _N // _C
    return pl.pallas_call(
        _tc_losses_body,
        grid=(grid,),
        in_specs=[
            pl.BlockSpec((15, _C), lambda i: (0, i)),
            pl.BlockSpec((1, _C), lambda i: (0, i)),
            pl.BlockSpec((15, _C), lambda i: (0, i)),
        ],
        out_specs=[
            pl.BlockSpec((3, _C), lambda i: (0, i)),
            pl.BlockSpec(memory_space=pltpu.SMEM),
        ],
        out_shape=[
            jax.ShapeDtypeStruct((3, _N), jnp.int32),
            jax.ShapeDtypeStruct((16,), jnp.int32),
        ],
    )(predT, labR, tgtT)


_U = 4


def _sc_select_body(keys_hbm, cnt_hbm, out_hbm,
                    keys_v, cnt_v, hist_v, merged_v, back_v, glob_v, out_stage,
                    cbuf_v, shared):
    f32, i32 = jnp.float32, jnp.int32
    cid = lax.axis_index("c")
    sid = lax.axis_index("s")
    lanes = lax.iota(i32, 16)
    zeros16 = jnp.zeros((16,), f32)

    a0 = jnp.where(cid == 0, 0, 2).astype(i32)
    a1 = jnp.where(cid == 0, 1, 0).astype(i32)
    base = sid * _CHUNK
    pltpu.sync_copy(keys_hbm.at[a0, pl.ds(base, _CHUNK)], keys_v.at[0])
    pltpu.sync_copy(keys_hbm.at[a1, pl.ds(base, _CHUNK)], keys_v.at[1])
    pltpu.sync_copy(cnt_hbm, cnt_v)

    cvec = cnt_v[...].astype(f32)

    def lane_of(vec, idx):
        return jnp.sum(jnp.where(lanes == idx, vec, 0.0))

    n = [lane_of(cvec, a0), lane_of(cvec, a1)]
    k = [(jnp.float32(0.7) * n[0]).astype(i32).astype(f32),
         (jnp.float32(0.7) * n[1]).astype(i32).astype(f32)]
    kk = [jnp.maximum(k[0], 1.0), jnp.maximum(k[1], 1.0)]
    prefix = [jnp.int32(0), jnp.int32(0)]

    lens = [jnp.int32(0), jnp.int32(0)]
    partial = [jnp.float32(0.0), jnp.float32(0.0)]

    for lvl in range(4):
        shift = 24 - 8 * lvl
        for slot in range(2):
            @plsc.parallel_loop(0, _U * _BINS // 16, unroll=_U)
            def _zero(v):
                hist_v[pl.ds(v * 16, 16)] = zeros16

            pfx = prefix[slot]
            if lvl == 0:
                @plsc.parallel_loop(0, _NV, unroll=_U)
                def _scan0(j, slot=slot):
                    key = keys_v[slot, pl.ds(j * 16, 16)]
                    digit = lax.shift_right_logical(key, 24)
                    cnts, last = plsc.scan_count(digit)
                    idx = (j & (_U - 1)) * _BINS + digit
                    plsc.addupdate_scatter(hist_v, [idx], cnts.astype(f32),
                                           mask=last)
            elif lvl == 1:
                def _scan1(j, carry, slot=slot, pfx=pfx):
                    wpos, acc = carry
                    key = keys_v[slot, pl.ds(j * 16, 16)]
                    hi = lax.shift_right_logical(key, 24)
                    m = hi == pfx
                    acc = acc + jnp.where(hi > pfx, plsc.bitcast(key, f32), 0.0)
                    digit = lax.shift_right_logical(key, 16) & 255
                    cnts, last = plsc.scan_count(digit, mask=m)
                    idx = (j & (_U - 1)) * _BINS + digit
                    plsc.addupdate_scatter(hist_v, [idx], cnts.astype(f32),
                                           mask=last)
                    plsc.store_compressed(cbuf_v.at[slot, pl.ds(wpos, 16)],
                                          key, mask=m)
                    nm = jnp.sum(jnp.where(m, 1.0, 0.0)).astype(i32)
                    return wpos + nm, acc
                wend, acc1 = plsc.parallel_loop(
                    0, _NV, unroll=_U,
                    carry=(jnp.int32(0), zeros16))(_scan1)
                cbuf_v[slot, pl.ds(wend, 16)] = jnp.zeros((16,), i32)
                lens[slot] = wend
                partial[slot] = jnp.sum(acc1)
            else:
                niter = lax.shift_right_logical(lens[slot] + 15, 4)

                def _scan23(j, slot=slot, pfx=pfx):
                    key = cbuf_v[slot, pl.ds(j * 16, 16)]
                    m = lax.shift_right_logical(key, shift + 8) == pfx
                    digit = lax.shift_right_logical(key, shift) & 255
                    cnts, last = plsc.scan_count(digit, mask=m)
                    idx = (j & (_U - 1)) * _BINS + digit
                    plsc.addupdate_scatter(hist_v, [idx], cnts.astype(f32),
                                           mask=last)
                plsc.parallel_loop(0, niter)(_scan23)

            @plsc.parallel_loop(0, _BINS // 16)
            def _merge(v, slot=slot):
                acc = zeros16
                for u in range(_U):
                    acc = acc + hist_v[pl.ds(u * _BINS + v * 16, 16)]
                merged_v[slot, pl.ds(v * 16, 16)] = acc

        pltpu.sync_copy(merged_v, shared.at[lvl & 1, sid])
        plsc.subcore_barrier()
        pltpu.sync_copy(shared.at[lvl & 1], back_v)

        for slot in range(2):
            @plsc.parallel_loop(0, _BINS // 16, unroll=2)
            def _gsum(v, slot=slot):
                acc = zeros16
                for t in range(_NT):
                    acc = acc + back_v[t, slot, pl.ds(v * 16, 16)]
                glob_v[slot, pl.ds(v * 16, 16)] = acc

            kk_s = kk[slot]

            def find_body(i, carry, slot=slot, kk_s=kk_s):
                best_d, abest, c_above = carry
                v = 15 - i
                h = glob_v[slot, pl.ds(v * 16, 16)]
                c_incl = plsc.cumsum(h)
                vtot = jnp.sum(h)
                c_geq = (vtot - c_incl + h) + c_above
                dig = (v * 16 + lanes).astype(f32)
                cand = jnp.where(c_geq >= kk_s, dig, -1.0)
                vmax = jnp.max(cand)
                aval = jnp.sum(jnp.where(dig == vmax, c_geq - h, 0.0))
                take = jnp.logical_and(best_d < 0.0, vmax >= 0.0)
                best_d = jnp.where(take, vmax, best_d)
                abest = jnp.where(take, aval, abest)
                return best_d, abest, c_above + vtot
            best_d, abest, _ = lax.fori_loop(
                0, 16, find_body, (jnp.float32(-1.0), jnp.float32(0.0), jnp.float32(0.0)))
            kk[slot] = kk_s - abest
            prefix[slot] = prefix[slot] * 256 + best_d.astype(i32)

    sums = []
    for slot in range(2):
        thr = prefix[slot]
        niter = lax.shift_right_logical(lens[slot] + 15, 4)

        def sum_body(j, acc, slot=slot, thr=thr):
            key = cbuf_v[slot, pl.ds(j * 16, 16)]
            vals = plsc.bitcast(key, f32)
            return acc + jnp.where(key > thr, vals, 0.0)
        accv = plsc.parallel_loop(0, niter, carry=zeros16)(sum_body)
        sums.append(jnp.sum(accv) + partial[slot])

    for slot in range(2):
        merged_v[slot, pl.ds(0, 16)] = jnp.where(lanes == 0, sums[slot], 0.0)
    pltpu.sync_copy(merged_v, shared.at[0, sid])
    plsc.subcore_barrier()
    pltpu.sync_copy(shared.at[0], back_v)

    def key_as_f32(t):
        return jnp.sum(plsc.bitcast(jnp.where(lanes == 0, t, 0).astype(i32), f32))

    res = []
    for slot in range(2):
        acc = zeros16
        for t in range(_NT):
            acc = acc + back_v[t, slot, pl.ds(0, 16)]
        tot = jnp.sum(acc)
        topk = tot + kk[slot] * key_as_f32(prefix[slot])
        res.append(jnp.where(k[slot] == 0.0, 0.0, topk))

    g_num = [jnp.where(n[s] == 0.0, 0.0, res[s]) for s in range(2)]
    g_den = [jnp.where(n[s] == 0.0, 1.0, k[s]) for s in range(2)]
    lane0num = jnp.where(cid == 0, res[0], g_num[0])
    lane0den = jnp.where(cid == 0, k[0], g_den[0])
    lane1num = jnp.where(cid == 0, g_num[1], 0.0)
    lane1den = jnp.where(cid == 0, g_den[1], 1.0)
    numv = jnp.where(lanes == 0, lane0num, jnp.where(lanes == 1, lane1num, 0.0))
    denv = jnp.where(lanes == 0, lane0den, jnp.where(lanes == 1, lane1den, 1.0))
    out_stage[...] = numv / denv

    @pl.when(sid == 0)
    def _():
        pltpu.sync_copy(out_stage, out_hbm.at[cid])


@functools.lru_cache(maxsize=1)
def _get_sc_select():
    return functools.partial(
        pl.kernel,
        out_type=jax.ShapeDtypeStruct((2, 16), jnp.float32),
        mesh=plsc.VectorSubcoreMesh(core_axis_name="c", subcore_axis_name="s",
                                    num_cores=2, num_subcores=_NT),
        compiler_params=pltpu.CompilerParams(needs_layout_passes=False),
        scratch_types=[
            pltpu.VMEM((2, _CHUNK), jnp.int32),
            pltpu.VMEM((16,), jnp.int32),
            pltpu.VMEM((_U * _BINS,), jnp.float32),
            pltpu.VMEM((2, _BINS), jnp.float32),
            pltpu.VMEM((_NT, 2, _BINS), jnp.float32),
            pltpu.VMEM((2, _BINS), jnp.float32),
            pltpu.VMEM((16,), jnp.float32),
            pltpu.VMEM((2, _CHUNK + 16), jnp.int32),
            pltpu.VMEM_SHARED((2, _NT, 2, _BINS), jnp.float32),
        ],
    )(_sc_select_body)


def kernel(pred, labels, offsets, landmarks):
    predT = pred.T
    labR = labels.reshape(1, _N)
    tgtT = jnp.concatenate(
        [jnp.zeros((1, _N), jnp.float32), offsets.T, landmarks.T], axis=0)
    keys, counts = _tc_losses(predT, labR, tgtT)
    out = _get_sc_select()(keys, counts)
    loss_cls = out[0, 0]
    loss_off = out[0, 1]
    loss_lm = out[1, 0]
    total = loss_cls + 0.5 * loss_off + 0.5 * loss_lm
    return (total, loss_cls, loss_off, loss_lm)


# speedup vs baseline: 1.0977x; 1.0977x over previous
import functools

import jax
import jax.numpy as jnp
from jax import lax
from jax.experimental import pallas as pl
from jax.experimental.pallas import tpu as pltpu
from jax.experimental.pallas import tpu_sc as plsc

_N = 131072
_C = 16384
_NT = 16
_CHUNK = _N // _NT
_NV = _CHUNK // 16
_BINS = 256
_LANEH = 16 * _BINS


def _tc_losses_body(pred_ref, lab_ref, off_ref, lm_ref, keys_ref, cnt_ref):
    i = pl.program_id(0)
    lab = lab_ref[...]
    s = jax.nn.sigmoid(pred_ref[0:1, :])
    y = jnp.where(jnp.logical_xor(lab == 1.0, lab == -2.0),
                  jnp.float32(1.0), jnp.float32(0.0))
    l_cls = jnp.maximum(s, 0.0) - s * y + jnp.log1p(jnp.exp(-jnp.abs(s)))
    m_cls = lab >= 0.0
    d = pred_ref[1:5, :] - off_ref[...]
    per_off = jnp.sum(d * d, axis=0, keepdims=True) / 4.0
    m_off = jnp.logical_xor(lab == 1.0, lab == -1.0)
    e = pred_ref[5:15, :] - lm_ref[...]
    per_lm = jnp.sum(e * e, axis=0, keepdims=True) / 10.0
    m_lm = lab == -2.0
    zero = jnp.int32(0)
    k0 = jnp.where(m_cls, lax.bitcast_convert_type(l_cls, jnp.int32), zero)
    k1 = jnp.where(m_off, lax.bitcast_convert_type(per_off, jnp.int32), zero)
    k2 = jnp.where(m_lm, lax.bitcast_convert_type(per_lm, jnp.int32), zero)
    keys_ref[...] = jnp.concatenate([k0, k1, k2], axis=0)

    @pl.when(i == 0)
    def _():
        for j in range(16):
            cnt_ref[j] = jnp.int32(0)

    cnt_ref[0] += jnp.sum(m_cls.astype(jnp.float32)).astype(jnp.int32)
    cnt_ref[1] += jnp.sum(m_off.astype(jnp.float32)).astype(jnp.int32)
    cnt_ref[2] += jnp.sum(m_lm.astype(jnp.float32)).astype(jnp.int32)


def _tc_losses(predT, labR, offT, lmT):
    grid = _N // _C
    return pl.pallas_call(
        _tc_losses_body,
        grid=(grid,),
        in_specs=[
            pl.BlockSpec((15, _C), lambda i: (0, i)),
            pl.BlockSpec((1, _C), lambda i: (0, i)),
            pl.BlockSpec((4, _C), lambda i: (0, i)),
            pl.BlockSpec((10, _C), lambda i: (0, i)),
        ],
        out_specs=[
            pl.BlockSpec((3, _C), lambda i: (0, i)),
            pl.BlockSpec(memory_space=pltpu.SMEM),
        ],
        out_shape=[
            jax.ShapeDtypeStruct((3, _N), jnp.int32),
            jax.ShapeDtypeStruct((16,), jnp.int32),
        ],
    )(predT, labR, offT, lmT)


_U = 4


def _sc_select_body(keys_hbm, cnt_hbm, out_hbm,
                    keys_v, cnt_v, hist_v, merged_v, back_v, glob_v, out_stage,
                    cbuf_v, shared):
    f32, i32 = jnp.float32, jnp.int32
    cid = lax.axis_index("c")
    sid = lax.axis_index("s")
    lanes = lax.iota(i32, 16)
    zeros16 = jnp.zeros((16,), f32)

    a0 = jnp.where(cid == 0, 0, 2).astype(i32)
    a1 = jnp.where(cid == 0, 1, 0).astype(i32)
    base = sid * _CHUNK
    pltpu.sync_copy(keys_hbm.at[a0, pl.ds(base, _CHUNK)], keys_v.at[0])
    pltpu.sync_copy(keys_hbm.at[a1, pl.ds(base, _CHUNK)], keys_v.at[1])
    pltpu.sync_copy(cnt_hbm, cnt_v)

    cvec = cnt_v[...].astype(f32)

    def lane_of(vec, idx):
        return jnp.sum(jnp.where(lanes == idx, vec, 0.0))

    n = [lane_of(cvec, a0), lane_of(cvec, a1)]
    k = [(jnp.float32(0.7) * n[0]).astype(i32).astype(f32),
         (jnp.float32(0.7) * n[1]).astype(i32).astype(f32)]
    kk = [jnp.maximum(k[0], 1.0), jnp.maximum(k[1], 1.0)]
    prefix = [jnp.int32(0), jnp.int32(0)]

    lens = [jnp.int32(0), jnp.int32(0)]

    for lvl in range(4):
        shift = 24 - 8 * lvl
        for slot in range(2):
            @plsc.parallel_loop(0, _U * _BINS // 16, unroll=_U)
            def _zero(v):
                hist_v[pl.ds(v * 16, 16)] = zeros16

            pfx = prefix[slot]
            if lvl == 0:
                @plsc.parallel_loop(0, _NV, unroll=_U)
                def _scan0(j, slot=slot):
                    key = keys_v[slot, pl.ds(j * 16, 16)]
                    digit = lax.shift_right_logical(key, 24)
                    cnts, last = plsc.scan_count(digit)
                    idx = (j & (_U - 1)) * _BINS + digit
                    plsc.addupdate_scatter(hist_v, [idx], cnts.astype(f32),
                                           mask=last)
            elif lvl == 1:
                def _scan1(j, wpos, slot=slot, pfx=pfx):
                    key = keys_v[slot, pl.ds(j * 16, 16)]
                    m = lax.shift_right_logical(key, 24) == pfx
                    digit = lax.shift_right_logical(key, 16) & 255
                    cnts, last = plsc.scan_count(digit, mask=m)
                    idx = (j & (_U - 1)) * _BINS + digit
                    plsc.addupdate_scatter(hist_v, [idx], cnts.astype(f32),
                                           mask=last)
                    plsc.store_compressed(cbuf_v.at[slot, pl.ds(wpos, 16)],
                                          key, mask=m)
                    nm = jnp.sum(jnp.where(m, 1.0, 0.0)).astype(i32)
                    return wpos + nm
                wend = plsc.parallel_loop(0, _NV, unroll=_U,
                                          carry=jnp.int32(0))(_scan1)
                cbuf_v[slot, pl.ds(wend, 16)] = jnp.zeros((16,), i32)
                lens[slot] = wend
            else:
                niter = lax.shift_right_logical(lens[slot] + 15, 4)

                def _scan23(j, slot=slot, pfx=pfx):
                    key = cbuf_v[slot, pl.ds(j * 16, 16)]
                    m = lax.shift_right_logical(key, shift + 8) == pfx
                    digit = lax.shift_right_logical(key, shift) & 255
                    cnts, last = plsc.scan_count(digit, mask=m)
                    idx = (j & (_U - 1)) * _BINS + digit
                    plsc.addupdate_scatter(hist_v, [idx], cnts.astype(f32),
                                           mask=last)
                plsc.parallel_loop(0, niter)(_scan23)

            @plsc.parallel_loop(0, _BINS // 16)
            def _merge(v, slot=slot):
                acc = zeros16
                for u in range(_U):
                    acc = acc + hist_v[pl.ds(u * _BINS + v * 16, 16)]
                merged_v[slot, pl.ds(v * 16, 16)] = acc

        pltpu.sync_copy(merged_v, shared.at[lvl & 1, sid])
        plsc.subcore_barrier()
        pltpu.sync_copy(shared.at[lvl & 1], back_v)

        for slot in range(2):
            @plsc.parallel_loop(0, _BINS // 16, unroll=2)
            def _gsum(v, slot=slot):
                acc = zeros16
                for t in range(_NT):
                    acc = acc + back_v[t, slot, pl.ds(v * 16, 16)]
                glob_v[slot, pl.ds(v * 16, 16)] = acc

            kk_s = kk[slot]

            def find_body(i, carry, slot=slot, kk_s=kk_s):
                best_d, abest, c_above = carry
                v = 15 - i
                h = glob_v[slot, pl.ds(v * 16, 16)]
                c_incl = plsc.cumsum(h)
                vtot = jnp.sum(h)
                c_geq = (vtot - c_incl + h) + c_above
                dig = (v * 16 + lanes).astype(f32)
                cand = jnp.where(c_geq >= kk_s, dig, -1.0)
                vmax = jnp.max(cand)
                aval = jnp.sum(jnp.where(dig == vmax, c_geq - h, 0.0))
                take = jnp.logical_and(best_d < 0.0, vmax >= 0.0)
                best_d = jnp.where(take, vmax, best_d)
                abest = jnp.where(take, aval, abest)
                return best_d, abest, c_above + vtot
            best_d, abest, _ = lax.fori_loop(
                0, 16, find_body, (jnp.float32(-1.0), jnp.float32(0.0), jnp.float32(0.0)))
            kk[slot] = kk_s - abest
            prefix[slot] = prefix[slot] * 256 + best_d.astype(i32)

    sums = []
    for slot in range(2):
        thr = prefix[slot]

        def sum_body(j, acc, slot=slot, thr=thr):
            key = keys_v[slot, pl.ds(j * 16, 16)]
            vals = plsc.bitcast(key, f32)
            return acc + jnp.where(key > thr, vals, 0.0)
        accv = plsc.parallel_loop(0, _NV, unroll=_U, carry=zeros16)(sum_body)
        sums.append(jnp.sum(accv))

    for slot in range(2):
        merged_v[slot, pl.ds(0, 16)] = jnp.where(lanes == 0, sums[slot], 0.0)
    pltpu.sync_copy(merged_v, shared.at[0, sid])
    plsc.subcore_barrier()
    pltpu.sync_copy(shared.at[0], back_v)

    def key_as_f32(t):
        return jnp.sum(plsc.bitcast(jnp.where(lanes == 0, t, 0).astype(i32), f32))

    res = []
    for slot in range(2):
        acc = zeros16
        for t in range(_NT):
            acc = acc + back_v[t, slot, pl.ds(0, 16)]
        tot = jnp.sum(acc)
        topk = tot + kk[slot] * key_as_f32(prefix[slot])
        res.append(jnp.where(k[slot] == 0.0, 0.0, topk))

    g_num = [jnp.where(n[s] == 0.0, 0.0, res[s]) for s in range(2)]
    g_den = [jnp.where(n[s] == 0.0, 1.0, k[s]) for s in range(2)]
    lane0num = jnp.where(cid == 0, res[0], g_num[0])
    lane0den = jnp.where(cid == 0, k[0], g_den[0])
    lane1num = jnp.where(cid == 0, g_num[1], 0.0)
    lane1den = jnp.where(cid == 0, g_den[1], 1.0)
    numv = jnp.where(lanes == 0, lane0num, jnp.where(lanes == 1, lane1num, 0.0))
    denv = jnp.where(lanes == 0, lane0den, jnp.where(lanes == 1, lane1den, 1.0))
    out_stage[...] = numv / denv

    @pl.when(sid == 0)
    def _():
        pltpu.sync_copy(out_stage, out_hbm.at[cid])


@functools.lru_cache(maxsize=1)
def _get_sc_select():
    return functools.partial(
        pl.kernel,
        out_type=jax.ShapeDtypeStruct((2, 16), jnp.float32),
        mesh=plsc.VectorSubcoreMesh(core_axis_name="c", subcore_axis_name="s",
                                    num_cores=2, num_subcores=_NT),
        compiler_params=pltpu.CompilerParams(needs_layout_passes=False),
        scratch_types=[
            pltpu.VMEM((2, _CHUNK), jnp.int32),
            pltpu.VMEM((16,), jnp.int32),
            pltpu.VMEM((_U * _BINS,), jnp.float32),
            pltpu.VMEM((2, _BINS), jnp.float32),
            pltpu.VMEM((_NT, 2, _BINS), jnp.float32),
            pltpu.VMEM((2, _BINS), jnp.float32),
            pltpu.VMEM((16,), jnp.float32),
            pltpu.VMEM((2, _CHUNK + 16), jnp.int32),
            pltpu.VMEM_SHARED((2, _NT, 2, _BINS), jnp.float32),
        ],
    )(_sc_select_body)


def kernel(pred, labels, offsets, landmarks):
    predT = pred.T
    labR = labels.reshape(1, _N)
    offT = offsets.T
    lmT = landmarks.T
    keys, counts = _tc_losses(predT, labR, offT, lmT)
    out = _get_sc_select()(keys, counts)
    loss_cls = out[0, 0]
    loss_off = out[0, 1]
    loss_lm = out[1, 0]
    total = loss_cls + 0.5 * loss_off + 0.5 * loss_lm
    return (total, loss_cls, loss_off, loss_lm)
